# baseline (device time: 185173 ns/iter reference)
import jax
import jax.numpy as jnp
from jax import lax
from jax.experimental import pallas as pl
from jax.experimental.pallas import tpu as pltpu

N_DEV = 8
N_HOP = N_DEV - 1
N_SLOT = 4
N_SUB = 2
W_TILE = 512


def _silu(y):
    return y * (1.0 / (1.0 + jnp.exp(-y)))


def kernel(x, w_mat):
    m_per, k = x.shape
    _, n_per = w_mat.shape
    kh = k // 2
    kq = kh // N_SUB

    def body(x_ref, w_hbm, out_hbm,
             cw_ref, ccw_ref, w_bf, w_stage, acc_ref, ostage_ref,
             send_cw, recv_cw, send_ccw, recv_ccw, credit_cw, credit_ccw,
             wdma_sem, odma_sem):
        my = lax.axis_index("i")
        left = lax.rem(my - 1 + N_DEV, N_DEV)
        right = lax.rem(my + 1, N_DEV)

        n_tiles = k // W_TILE
        wdmas = [None] * n_tiles

        def wtile(t):
            return pltpu.make_async_copy(
                w_hbm.at[pl.ds(t * W_TILE, W_TILE), :],
                w_stage.at[t % 2],
                wdma_sem.at[t % 2],
            )

        wdmas[0] = wtile(0)
        wdmas[0].start()

        for s in range(N_SUB):
            cw_ref[0, s] = x_ref[:, s * kq:(s + 1) * kq].astype(jnp.bfloat16)
            ccw_ref[0, s] = x_ref[:, kh + s * kq:kh + (s + 1) * kq].astype(
                jnp.bfloat16)

        barrier_sem = pltpu.get_barrier_semaphore()
        for nbr in (left, right):
            pl.semaphore_signal(
                barrier_sem, inc=1,
                device_id=(nbr,), device_id_type=pl.DeviceIdType.MESH,
            )
        pl.semaphore_wait(barrier_sem, 2)

        def make(h, s, cw):
            a = h % N_SLOT
            b = (h + 1) % N_SLOT
            i = N_SUB * h + s
            if cw:
                return pltpu.make_async_remote_copy(
                    src_ref=cw_ref.at[a, s], dst_ref=cw_ref.at[b, s],
                    send_sem=send_cw.at[i], recv_sem=recv_cw.at[i],
                    device_id=(right,), device_id_type=pl.DeviceIdType.MESH,
                )
            return pltpu.make_async_remote_copy(
                src_ref=ccw_ref.at[a, s], dst_ref=ccw_ref.at[b, s],
                send_sem=send_ccw.at[i], recv_sem=recv_ccw.at[i],
                device_id=(left,), device_id_type=pl.DeviceIdType.MESH,
            )

        d_cw = [None] * (N_SUB * N_HOP)
        d_ccw = [None] * (N_SUB * N_HOP)
        for s in range(N_SUB):
            d_cw[s] = make(0, s, True)
            d_cw[s].start()
            d_ccw[s] = make(0, s, False)
            d_ccw[s].start()

        for t in range(n_tiles):
            wdmas[t].wait()
            if t + 1 < n_tiles:
                wdmas[t + 1] = wtile(t + 1)
                wdmas[t + 1].start()
            w_bf[pl.ds(t * W_TILE, W_TILE), :] = (
                w_stage[t % 2].astype(jnp.bfloat16))

        out_dmas = []

        def push_out(src_ref, origin, sem_idx):
            dma = pltpu.make_async_copy(
                src_ref,
                out_hbm.at[pl.ds(origin * m_per, m_per), :],
                odma_sem.at[sem_idx],
            )
            dma.start()
            out_dmas.append(dma)

        def wq(cw, s):
            base = (0 if cw else kh) + s * kq
            return w_bf[pl.ds(base, kq), :]

        for h in range(N_HOP):
            r = (h + 1) % N_SLOT
            j = 6 - h
            p_cw = None
            p_ccw = None
            for s in range(N_SUB):
                d_cw[N_SUB * h + s].wait_recv()
                d_ccw[N_SUB * h + s].wait_recv()
                if h < N_HOP - 1:
                    if s == 0 and h + 1 >= N_SLOT - 1:
                        pl.semaphore_wait(credit_cw, 1)
                        pl.semaphore_wait(credit_ccw, 1)
                    d_cw[N_SUB * (h + 1) + s] = make(h + 1, s, True)
                    d_cw[N_SUB * (h + 1) + s].start()
                    d_ccw[N_SUB * (h + 1) + s] = make(h + 1, s, False)
                    d_ccw[N_SUB * (h + 1) + s].start()
                pc = jnp.dot(cw_ref[r, s], wq(True, s),
                             preferred_element_type=jnp.float32)
                qc = jnp.dot(ccw_ref[r, s], wq(False, s),
                             preferred_element_type=jnp.float32)
                if h > 3 and s == 0:
                    pc = pc + acc_ref[2 * j + 1]
                    qc = qc + acc_ref[2 * j]
                p_cw = pc if p_cw is None else p_cw + pc
                p_ccw = qc if p_ccw is None else p_ccw + qc

            o_cw = lax.rem(my - h - 1 + N_DEV, N_DEV)
            o_ccw = lax.rem(my + h + 1, N_DEV)
            if h < 3:
                acc_ref[2 * h] = p_cw
                acc_ref[2 * h + 1] = p_ccw
            elif h == 3:
                ostage_ref[1] = _silu(p_cw + p_ccw)
                push_out(ostage_ref.at[1], o_cw, 1)
            else:
                acc_ref[2 * j + 1] = _silu(p_cw)
                push_out(acc_ref.at[2 * j + 1], o_cw, 2 * h - 6)
                acc_ref[2 * j] = _silu(p_ccw)
                push_out(acc_ref.at[2 * j], o_ccw, 2 * h - 5)

            if h == 0:
                y = jnp.dot(cw_ref[0, 0], wq(True, 0),
                            preferred_element_type=jnp.float32)
                for s in range(1, N_SUB):
                    y = y + jnp.dot(cw_ref[0, s], wq(True, s),
                                    preferred_element_type=jnp.float32)
                for s in range(N_SUB):
                    y = y + jnp.dot(ccw_ref[0, s], wq(False, s),
                                    preferred_element_type=jnp.float32)
                ostage_ref[0] = _silu(y)
                push_out(ostage_ref.at[0], my, 0)

            for s in range(N_SUB):
                d_cw[N_SUB * h + s].wait_send()
                d_ccw[N_SUB * h + s].wait_send()
            if h <= N_SLOT - 1:
                pl.semaphore_signal(
                    credit_cw, inc=1,
                    device_id=(left,), device_id_type=pl.DeviceIdType.MESH,
                )
                pl.semaphore_signal(
                    credit_ccw, inc=1,
                    device_id=(right,), device_id_type=pl.DeviceIdType.MESH,
                )

        for dma in out_dmas:
            dma.wait()

    return pl.pallas_call(
        body,
        out_shape=jax.ShapeDtypeStruct((N_DEV * m_per, n_per), jnp.float32),
        in_specs=[
            pl.BlockSpec(memory_space=pltpu.VMEM),
            pl.BlockSpec(memory_space=pltpu.MemorySpace.HBM),
        ],
        out_specs=pl.BlockSpec(memory_space=pltpu.MemorySpace.HBM),
        scratch_shapes=[
            pltpu.VMEM((N_SLOT, N_SUB, m_per, kq), jnp.bfloat16),
            pltpu.VMEM((N_SLOT, N_SUB, m_per, kq), jnp.bfloat16),
            pltpu.VMEM((k, n_per), jnp.bfloat16),
            pltpu.VMEM((2, W_TILE, n_per), jnp.float32),
            pltpu.VMEM((6, m_per, n_per), jnp.float32),
            pltpu.VMEM((2, m_per, n_per), jnp.float32),
            pltpu.SemaphoreType.DMA((N_SUB * N_HOP,)),
            pltpu.SemaphoreType.DMA((N_SUB * N_HOP,)),
            pltpu.SemaphoreType.DMA((N_SUB * N_HOP,)),
            pltpu.SemaphoreType.DMA((N_SUB * N_HOP,)),
            pltpu.SemaphoreType.REGULAR,
            pltpu.SemaphoreType.REGULAR,
            pltpu.SemaphoreType.DMA((2,)),
            pltpu.SemaphoreType.DMA((8,)),
        ],
        compiler_params=pltpu.CompilerParams(
            collective_id=0,
            vmem_limit_bytes=100 * 1024 * 1024,
        ),
    )(x, w_mat)


# device time: 184186 ns/iter; 1.0054x vs baseline; 1.0054x over previous
import jax
import jax.numpy as jnp
from jax import lax
from jax.experimental import pallas as pl
from jax.experimental.pallas import tpu as pltpu

N_DEV = 8
N_HOP = N_DEV - 1
N_SLOT = 4
N_SUB = 4
W_TILE = 512


def _silu(y):
    return y * (1.0 / (1.0 + jnp.exp(-y)))


def kernel(x, w_mat):
    m_per, k = x.shape
    _, n_per = w_mat.shape
    kh = k // 2
    kq = kh // N_SUB

    def body(x_ref, w_hbm, out_hbm,
             cw_ref, ccw_ref, w_bf, w_stage, acc_ref, ostage_ref,
             send_cw, recv_cw, send_ccw, recv_ccw, credit_cw, credit_ccw,
             wdma_sem, odma_sem):
        my = lax.axis_index("i")
        left = lax.rem(my - 1 + N_DEV, N_DEV)
        right = lax.rem(my + 1, N_DEV)

        n_tiles = k // W_TILE
        wdmas = [None] * n_tiles

        def wtile(t):
            return pltpu.make_async_copy(
                w_hbm.at[pl.ds(t * W_TILE, W_TILE), :],
                w_stage.at[t % 2],
                wdma_sem.at[t % 2],
            )

        wdmas[0] = wtile(0)
        wdmas[0].start()

        for s in range(N_SUB):
            cw_ref[0, s] = x_ref[:, s * kq:(s + 1) * kq].astype(jnp.bfloat16)
            ccw_ref[0, s] = x_ref[:, kh + s * kq:kh + (s + 1) * kq].astype(
                jnp.bfloat16)

        barrier_sem = pltpu.get_barrier_semaphore()
        for nbr in (left, right):
            pl.semaphore_signal(
                barrier_sem, inc=1,
                device_id=(nbr,), device_id_type=pl.DeviceIdType.MESH,
            )
        pl.semaphore_wait(barrier_sem, 2)

        def make(h, s, cw):
            a = h % N_SLOT
            b = (h + 1) % N_SLOT
            i = (h % N_SLOT) * N_SUB + s
            if cw:
                return pltpu.make_async_remote_copy(
                    src_ref=cw_ref.at[a, s], dst_ref=cw_ref.at[b, s],
                    send_sem=send_cw.at[i], recv_sem=recv_cw.at[i],
                    device_id=(right,), device_id_type=pl.DeviceIdType.MESH,
                )
            return pltpu.make_async_remote_copy(
                src_ref=ccw_ref.at[a, s], dst_ref=ccw_ref.at[b, s],
                send_sem=send_ccw.at[i], recv_sem=recv_ccw.at[i],
                device_id=(left,), device_id_type=pl.DeviceIdType.MESH,
            )

        d_cw = [None] * (N_SUB * N_HOP)
        d_ccw = [None] * (N_SUB * N_HOP)
        for s in range(N_SUB):
            d_cw[s] = make(0, s, True)
            d_cw[s].start()
            d_ccw[s] = make(0, s, False)
            d_ccw[s].start()

        for t in range(n_tiles):
            wdmas[t].wait()
            if t + 1 < n_tiles:
                wdmas[t + 1] = wtile(t + 1)
                wdmas[t + 1].start()
            w_bf[pl.ds(t * W_TILE, W_TILE), :] = (
                w_stage[t % 2].astype(jnp.bfloat16))

        out_dmas = []

        def push_out(src_ref, origin, sem_idx):
            dma = pltpu.make_async_copy(
                src_ref,
                out_hbm.at[pl.ds(origin * m_per, m_per), :],
                odma_sem.at[sem_idx],
            )
            dma.start()
            out_dmas.append(dma)

        def wq(cw, s):
            base = (0 if cw else kh) + s * kq
            return w_bf[pl.ds(base, kq), :]

        for h in range(N_HOP):
            r = (h + 1) % N_SLOT
            j = 6 - h
            p_cw = None
            p_ccw = None
            for s in range(N_SUB):
                d_cw[N_SUB * h + s].wait_recv()
                d_ccw[N_SUB * h + s].wait_recv()
                if h < N_HOP - 1:
                    if s == 0 and h + 1 >= N_SLOT - 1:
                        pl.semaphore_wait(credit_cw, 1)
                        pl.semaphore_wait(credit_ccw, 1)
                    d_cw[N_SUB * (h + 1) + s] = make(h + 1, s, True)
                    d_cw[N_SUB * (h + 1) + s].start()
                    d_ccw[N_SUB * (h + 1) + s] = make(h + 1, s, False)
                    d_ccw[N_SUB * (h + 1) + s].start()
                pc = jnp.dot(cw_ref[r, s], wq(True, s),
                             preferred_element_type=jnp.float32)
                qc = jnp.dot(ccw_ref[r, s], wq(False, s),
                             preferred_element_type=jnp.float32)
                if h > 3 and s == 0:
                    pc = pc + acc_ref[2 * j + 1]
                    qc = qc + acc_ref[2 * j]
                p_cw = pc if p_cw is None else p_cw + pc
                p_ccw = qc if p_ccw is None else p_ccw + qc

            o_cw = lax.rem(my - h - 1 + N_DEV, N_DEV)
            o_ccw = lax.rem(my + h + 1, N_DEV)
            if h < 3:
                acc_ref[2 * h] = p_cw
                acc_ref[2 * h + 1] = p_ccw
            elif h == 3:
                ostage_ref[1] = _silu(p_cw + p_ccw)
                push_out(ostage_ref.at[1], o_cw, 1)
            else:
                acc_ref[2 * j + 1] = _silu(p_cw)
                push_out(acc_ref.at[2 * j + 1], o_cw, 2 * h - 6)
                acc_ref[2 * j] = _silu(p_ccw)
                push_out(acc_ref.at[2 * j], o_ccw, 2 * h - 5)

            if h == 0:
                y = jnp.dot(cw_ref[0, 0], wq(True, 0),
                            preferred_element_type=jnp.float32)
                for s in range(1, N_SUB):
                    y = y + jnp.dot(cw_ref[0, s], wq(True, s),
                                    preferred_element_type=jnp.float32)
                for s in range(N_SUB):
                    y = y + jnp.dot(ccw_ref[0, s], wq(False, s),
                                    preferred_element_type=jnp.float32)
                ostage_ref[0] = _silu(y)
                push_out(ostage_ref.at[0], my, 0)

            for s in range(N_SUB):
                d_cw[N_SUB * h + s].wait_send()
                d_ccw[N_SUB * h + s].wait_send()
            if h <= N_SLOT - 1:
                pl.semaphore_signal(
                    credit_cw, inc=1,
                    device_id=(left,), device_id_type=pl.DeviceIdType.MESH,
                )
                pl.semaphore_signal(
                    credit_ccw, inc=1,
                    device_id=(right,), device_id_type=pl.DeviceIdType.MESH,
                )

        for dma in out_dmas:
            dma.wait()

    return pl.pallas_call(
        body,
        out_shape=jax.ShapeDtypeStruct((N_DEV * m_per, n_per), jnp.float32),
        in_specs=[
            pl.BlockSpec(memory_space=pltpu.VMEM),
            pl.BlockSpec(memory_space=pltpu.MemorySpace.HBM),
        ],
        out_specs=pl.BlockSpec(memory_space=pltpu.MemorySpace.HBM),
        scratch_shapes=[
            pltpu.VMEM((N_SLOT, N_SUB, m_per, kq), jnp.bfloat16),
            pltpu.VMEM((N_SLOT, N_SUB, m_per, kq), jnp.bfloat16),
            pltpu.VMEM((k, n_per), jnp.bfloat16),
            pltpu.VMEM((2, W_TILE, n_per), jnp.float32),
            pltpu.VMEM((6, m_per, n_per), jnp.float32),
            pltpu.VMEM((2, m_per, n_per), jnp.float32),
            pltpu.SemaphoreType.DMA((N_SUB * N_SLOT,)),
            pltpu.SemaphoreType.DMA((N_SUB * N_SLOT,)),
            pltpu.SemaphoreType.DMA((N_SUB * N_SLOT,)),
            pltpu.SemaphoreType.DMA((N_SUB * N_SLOT,)),
            pltpu.SemaphoreType.REGULAR,
            pltpu.SemaphoreType.REGULAR,
            pltpu.SemaphoreType.DMA((2,)),
            pltpu.SemaphoreType.DMA((8,)),
        ],
        compiler_params=pltpu.CompilerParams(
            collective_id=0,
            vmem_limit_bytes=100 * 1024 * 1024,
        ),
    )(x, w_mat)
